# Initial kernel scaffold; baseline (speedup 1.0000x reference)
import jax
import jax.numpy as jnp
from jax.experimental import pallas as pl


def kernel(x, W):
    raise NotImplementedError("write your pallas kernel here")


# speedup vs baseline: 1.9759x; 1.9759x over previous
import jax
import jax.numpy as jnp
from jax.experimental import pallas as pl

_N_TOKENS = 32768
_D = 768
_E = 8
_BLK = 2048


def _router_body(x_ref, wt_ref, w_out_ref, i_out_ref):
    s = jnp.dot(x_ref[...], wt_ref[...], preferred_element_type=jnp.float32)
    m = jnp.max(s, axis=1, keepdims=True)
    lane = jax.lax.broadcasted_iota(jnp.int32, s.shape, 1)
    idx = jnp.min(jnp.where(s == m, lane, _E), axis=1, keepdims=True)
    w_out_ref[...] = m
    i_out_ref[...] = idx


def kernel(x, W):
    wt = W.T
    grid = (_N_TOKENS // _BLK,)
    weights, indices = pl.pallas_call(
        _router_body,
        grid=grid,
        in_specs=[
            pl.BlockSpec((_BLK, _D), lambda i: (i, 0)),
            pl.BlockSpec((_D, _E), lambda i: (0, 0)),
        ],
        out_specs=[
            pl.BlockSpec((_BLK, 1), lambda i: (i, 0)),
            pl.BlockSpec((_BLK, 1), lambda i: (i, 0)),
        ],
        out_shape=[
            jax.ShapeDtypeStruct((_N_TOKENS, 1), jnp.float32),
            jax.ShapeDtypeStruct((_N_TOKENS, 1), jnp.int32),
        ],
    )(x, wt)
    return (weights, indices)


# speedup vs baseline: 2.0603x; 1.0427x over previous
import jax
import jax.numpy as jnp
from jax.experimental import pallas as pl

_N_TOKENS = 32768
_D = 768
_E = 8
_BLK = 4096


def _router_body(x_ref, wt_ref, w_out_ref, i_out_ref):
    s = jnp.dot(x_ref[...], wt_ref[...], preferred_element_type=jnp.float32)
    m = jnp.max(s, axis=1, keepdims=True)
    lane = jax.lax.broadcasted_iota(jnp.int32, s.shape, 1)
    idx = jnp.min(jnp.where(s == m, lane, _E), axis=1, keepdims=True)
    w_out_ref[...] = m
    i_out_ref[...] = idx


def kernel(x, W):
    wt = W.T
    grid = (_N_TOKENS // _BLK,)
    weights, indices = pl.pallas_call(
        _router_body,
        grid=grid,
        in_specs=[
            pl.BlockSpec((_BLK, _D), lambda i: (i, 0)),
            pl.BlockSpec((_D, _E), lambda i: (0, 0)),
        ],
        out_specs=[
            pl.BlockSpec((_BLK, 1), lambda i: (i, 0)),
            pl.BlockSpec((_BLK, 1), lambda i: (i, 0)),
        ],
        out_shape=[
            jax.ShapeDtypeStruct((_N_TOKENS, 1), jnp.float32),
            jax.ShapeDtypeStruct((_N_TOKENS, 1), jnp.int32),
        ],
    )(x, wt)
    return (weights, indices)
